# baseline (speedup 1.0000x reference)
import functools

import jax
import jax.numpy as jnp
from jax import lax
from jax.experimental import pallas as pl
from jax.experimental.pallas import tpu as pltpu
from jax.experimental.pallas import tpu_sc as plsc

D_MODEL = 4096
NUM_EXPERTS = 64
TOP_K = 8
Z_LOSS = 0.001

BT = 1024
T_TOTAL = 8192
NW = 32
TPW = T_TOTAL // NW
L = 16


def _tc_block(x_ref, w_ref, logits_ref, stats_ref, fin_ref):
    step = pl.program_id(0)
    nsteps = pl.num_programs(0)

    h = x_ref[...]
    w = w_ref[...]
    logits = jax.lax.dot_general(
        h, w, (((1,), (1,)), ((), ())), preferred_element_type=jnp.float32
    )
    logits_ref[...] = logits

    m = jnp.max(logits, axis=-1)
    lm = logits - m[:, None]
    e = jnp.exp(lm)
    s = jnp.sum(e, axis=-1)
    logs = jnp.log(s)
    ent = logs - jnp.sum(e * lm, axis=-1) / s
    z = m + logs

    part = jnp.stack(
        [
            jnp.sum(ent),
            jnp.min(ent),
            jnp.sum(z * z),
            jnp.sum(logits * logits),
            0.0,
            0.0,
            0.0,
            0.0,
        ]
    )[None, :]

    @pl.when(step == 0)
    def _():
        stats_ref[...] = part

    @pl.when(step != 0)
    def _():
        old = stats_ref[...]
        lane = jax.lax.broadcasted_iota(jnp.int32, old.shape, 1)
        stats_ref[...] = jnp.where(lane == 1, jnp.minimum(old, part), old + part)

    @pl.when(step == nsteps - 1)
    def _():
        t_tot = jnp.float32(nsteps * BT)
        stats = stats_ref[0, :]
        fin_ref[...] = jnp.stack(
            [
                stats[0] / t_tot,
                stats[1],
                Z_LOSS * stats[2] / t_tot,
                jnp.sqrt(stats[3] / (t_tot * NUM_EXPERTS)),
                0.0,
                0.0,
                0.0,
                0.0,
            ]
        )[None, :]


_sc_mesh = plsc.VectorSubcoreMesh(core_axis_name="c", subcore_axis_name="s")


@functools.partial(
    pl.kernel,
    mesh=_sc_mesh,
    out_type=[
        jax.ShapeDtypeStruct((T_TOTAL, L), jnp.int32),
        jax.ShapeDtypeStruct((T_TOTAL, L), jnp.float32),
        jax.ShapeDtypeStruct((NW, NUM_EXPERTS), jnp.float32),
        jax.ShapeDtypeStruct((NW, L), jnp.float32),
    ],
    scratch_types=[
        pltpu.VMEM((TPW, NUM_EXPERTS), jnp.float32),
        pltpu.VMEM((TPW, L), jnp.int32),
        pltpu.VMEM((TPW, L), jnp.float32),
        pltpu.VMEM((NUM_EXPERTS,), jnp.float32),
        pltpu.VMEM((L,), jnp.float32),
    ],
)
def _sc_topk(logits_hbm, idx_hbm, w_hbm, cnt_hbm, acc_hbm,
             lbuf, ibuf, wbuf, cbuf, abuf):
    wid = lax.axis_index("s") * 2 + lax.axis_index("c")
    base = wid * TPW
    pltpu.sync_copy(logits_hbm.at[pl.ds(base, TPW)], lbuf)

    iota = lax.iota(jnp.int32, L)
    minf = jnp.full((L,), -jnp.inf, jnp.float32)
    zero = jnp.zeros((L,), jnp.float32)
    perms = [iota ^ k for k in (8, 4, 2, 1)]

    def _ball(v, op):
        for p in perms:
            v = op(v, v[p])
        return v

    def token_body(t, carry):
        cacc0, cacc1, cacc2, cacc3, aacc = carry
        c0 = lbuf[t, pl.ds(0, L)]
        c1 = lbuf[t, pl.ds(L, L)]
        c2 = lbuf[t, pl.ds(2 * L, L)]
        c3 = lbuf[t, pl.ds(3 * L, L)]
        o0, o1, o2, o3 = c0, c1, c2, c3

        widx = jnp.zeros((L,), jnp.int32)
        wval = zero
        m0b = zero
        cs = (c0, c1, c2, c3)
        for r in range(TOP_K):
            c0, c1, c2, c3 = cs
            mvb = _ball(jnp.maximum(jnp.maximum(c0, c1), jnp.maximum(c2, c3)),
                        jnp.maximum)
            i0 = jnp.where(c0 == mvb, iota, NUM_EXPERTS)
            i1 = jnp.where(c1 == mvb, iota + L, NUM_EXPERTS)
            i2 = jnp.where(c2 == mvb, iota + 2 * L, NUM_EXPERTS)
            i3 = jnp.where(c3 == mvb, iota + 3 * L, NUM_EXPERTS)
            imb = _ball(jnp.minimum(jnp.minimum(i0, i1), jnp.minimum(i2, i3)),
                        jnp.minimum)
            w0, w1, w2, w3 = i0 == imb, i1 == imb, i2 == imb, i3 == imb
            cs = (
                jnp.where(w0, minf, c0),
                jnp.where(w1, minf, c1),
                jnp.where(w2, minf, c2),
                jnp.where(w3, minf, c3),
            )
            widx = jnp.where(iota == r, imb, widx)
            wval = jnp.where(iota == r, mvb, wval)
            if r == 0:
                m0b = mvb
                one = jnp.ones((L,), jnp.float32)
                cacc0 = cacc0 + jnp.where(w0, one, zero)
                cacc1 = cacc1 + jnp.where(w1, one, zero)
                cacc2 = cacc2 + jnp.where(w2, one, zero)
                cacc3 = cacc3 + jnp.where(w3, one, zero)

        srow = _ball(
            jnp.exp(o0 - m0b) + jnp.exp(o1 - m0b)
            + jnp.exp(o2 - m0b) + jnp.exp(o3 - m0b),
            jnp.add,
        )
        p = jnp.exp(wval - m0b) / srow
        p = jnp.where(iota < TOP_K, p, zero)
        psum = _ball(p, jnp.add)
        pn = p / (psum + 1e-9)
        ibuf[t, :] = widx
        wbuf[t, :] = pn
        return (cacc0, cacc1, cacc2, cacc3, aacc + pn)

    init = (zero, zero, zero, zero, zero)
    cacc0, cacc1, cacc2, cacc3, aacc = lax.fori_loop(0, TPW, token_body, init)

    cbuf[pl.ds(0, L)] = cacc0
    cbuf[pl.ds(L, L)] = cacc1
    cbuf[pl.ds(2 * L, L)] = cacc2
    cbuf[pl.ds(3 * L, L)] = cacc3
    abuf[...] = aacc
    pltpu.sync_copy(ibuf, idx_hbm.at[pl.ds(base, TPW)])
    pltpu.sync_copy(wbuf, w_hbm.at[pl.ds(base, TPW)])
    pltpu.sync_copy(cbuf, cnt_hbm.at[wid])
    pltpu.sync_copy(abuf, acc_hbm.at[wid])


def kernel(x, W):
    B, S, D = x.shape
    T = B * S
    h = x.reshape(T, D)
    nsteps = T // BT

    logits, _, fin = pl.pallas_call(
        _tc_block,
        grid=(nsteps,),
        in_specs=[
            pl.BlockSpec((BT, D), lambda i: (i, 0)),
            pl.BlockSpec((NUM_EXPERTS, D), lambda i: (0, 0)),
        ],
        out_specs=[
            pl.BlockSpec((BT, NUM_EXPERTS), lambda i: (i, 0)),
            pl.BlockSpec((1, 8), lambda i: (0, 0)),
            pl.BlockSpec((1, 8), lambda i: (0, 0)),
        ],
        out_shape=[
            jax.ShapeDtypeStruct((T, NUM_EXPERTS), jnp.float32),
            jax.ShapeDtypeStruct((1, 8), jnp.float32),
            jax.ShapeDtypeStruct((1, 8), jnp.float32),
        ],
    )(h, W)

    idx_pad, w_pad, cnt_p, acc_p = _sc_topk(logits)

    topi = idx_pad[:, :TOP_K]
    topw = w_pad[:, :TOP_K]
    counts = jnp.sum(cnt_p, axis=0)
    wsums = jnp.sum(acc_p, axis=0)
    t_tot = jnp.float32(T)
    cmean = counts.mean()
    cv = counts.std() / (cmean + 1e-9)
    margin = (wsums[0] - wsums[1]) / t_tot
    conf = wsums[0] / t_tot

    return (
        topi.astype(jnp.int64),
        topw,
        fin[0, 0],
        fin[0, 1],
        cv,
        counts,
        fin[0, 2],
        fin[0, 3],
        margin,
        conf,
    )


# speedup vs baseline: 1.7497x; 1.7497x over previous
import jax
import jax.numpy as jnp
from jax.experimental import pallas as pl

D_MODEL = 4096
NUM_EXPERTS = 64
TOP_K = 8
Z_LOSS = 0.001

BT = 512


def _router_block(x_ref, w_ref, idx_ref, wts_ref, counts_ref, stats_ref, fin_ref):
    step = pl.program_id(0)
    nsteps = pl.num_programs(0)

    h = x_ref[...]
    w = w_ref[...]
    logits = jax.lax.dot_general(
        h, w, (((1,), (1,)), ((), ())), preferred_element_type=jnp.float32
    )

    coli = jax.lax.broadcasted_iota(jnp.int32, logits.shape, 1)
    colf = coli.astype(jnp.float32)
    lcur = logits
    vals = []
    idxf = []
    for _ in range(TOP_K):
        vk = jnp.max(lcur, axis=-1)
        ik = jnp.min(jnp.where(lcur == vk[:, None], colf, jnp.float32(NUM_EXPERTS)),
                     axis=-1)
        vals.append(vk)
        idxf.append(ik)
        lcur = jnp.where(colf == ik[:, None], -jnp.inf, lcur)

    m = vals[0]
    lm = logits - m[:, None]
    e = jnp.exp(lm)
    s = jnp.sum(e, axis=-1)
    logs = jnp.log(s)
    ent = logs - jnp.sum(e * lm, axis=-1) / s
    z = m + logs

    inv_s = 1.0 / s
    tis = [ik.astype(jnp.int32) for ik in idxf]
    ps = [jnp.exp(v - m) * inv_s for v in vals]
    psum = ps[0]
    for k in range(1, TOP_K):
        psum = psum + ps[k]
    r = 1.0 / (psum + 1e-9)
    ws = [p * r for p in ps]

    idx_ref[...] = jnp.stack(tis, axis=0)
    wts_ref[...] = jnp.stack(ws, axis=0)

    onehot = (coli == tis[0][:, None]).astype(jnp.float32)
    cnt = jnp.sum(onehot, axis=0)[None, :]

    part = jnp.stack(
        [
            jnp.sum(ent),
            jnp.min(ent),
            jnp.sum(z * z),
            jnp.sum(logits * logits),
            jnp.sum(ws[0] - ws[1]),
            jnp.sum(ws[0]),
            0.0,
            0.0,
        ]
    )[None, :]

    @pl.when(step == 0)
    def _():
        counts_ref[...] = cnt
        stats_ref[...] = part

    @pl.when(step != 0)
    def _():
        counts_ref[...] += cnt
        old = stats_ref[...]
        lane = jax.lax.broadcasted_iota(jnp.int32, old.shape, 1)
        stats_ref[...] = jnp.where(lane == 1, jnp.minimum(old, part), old + part)

    @pl.when(step == nsteps - 1)
    def _():
        t_tot = jnp.float32(nsteps * BT)
        counts = counts_ref[0, :]
        stats = stats_ref[0, :]
        cmean = jnp.sum(counts) / NUM_EXPERTS
        cstd = jnp.sqrt(jnp.sum((counts - cmean) ** 2) / NUM_EXPERTS)
        cv = cstd / (cmean + 1e-9)
        fin_ref[...] = jnp.stack(
            [
                stats[0] / t_tot,
                stats[1],
                cv,
                Z_LOSS * stats[2] / t_tot,
                jnp.sqrt(stats[3] / (t_tot * NUM_EXPERTS)),
                stats[4] / t_tot,
                stats[5] / t_tot,
                0.0,
            ]
        )[None, :]


def kernel(x, W):
    B, S, D = x.shape
    T = B * S
    h = x.reshape(T, D)
    nsteps = T // BT

    topi_t, topw_t, counts, _, fin = pl.pallas_call(
        _router_block,
        grid=(nsteps,),
        in_specs=[
            pl.BlockSpec((BT, D), lambda i: (i, 0)),
            pl.BlockSpec((NUM_EXPERTS, D), lambda i: (0, 0)),
        ],
        out_specs=[
            pl.BlockSpec((TOP_K, BT), lambda i: (0, i)),
            pl.BlockSpec((TOP_K, BT), lambda i: (0, i)),
            pl.BlockSpec((1, NUM_EXPERTS), lambda i: (0, 0)),
            pl.BlockSpec((1, 8), lambda i: (0, 0)),
            pl.BlockSpec((1, 8), lambda i: (0, 0)),
        ],
        out_shape=[
            jax.ShapeDtypeStruct((TOP_K, T), jnp.int32),
            jax.ShapeDtypeStruct((TOP_K, T), jnp.float32),
            jax.ShapeDtypeStruct((1, NUM_EXPERTS), jnp.float32),
            jax.ShapeDtypeStruct((1, 8), jnp.float32),
            jax.ShapeDtypeStruct((1, 8), jnp.float32),
        ],
    )(h, W)

    return (
        topi_t.T.astype(jnp.int64),
        topw_t.T,
        fin[0, 0],
        fin[0, 1],
        fin[0, 2],
        counts[0],
        fin[0, 3],
        fin[0, 4],
        fin[0, 5],
        fin[0, 6],
    )
